# Initial kernel scaffold; baseline (speedup 1.0000x reference)
import functools

import jax
import jax.numpy as jnp
from jax.experimental import pallas as pl
from jax.experimental.pallas import tpu as pltpu

N_USER = 25000
N_ITEM = 25000
N = N_USER + N_ITEM
NNZ = 800000
EMB = 80
B = 4096
N_LAYER = 3
EMB_RATIO = 0.5

ROW_BLK = 2500


def _dense_layer_body(e_ref, l_ref, w1_ref, w2_ref, bias_ref, out_ref):
    e = e_ref[...]
    l = l_ref[...]
    acc = jnp.dot(l + e, w1_ref[...], preferred_element_type=jnp.float32)
    acc += jnp.dot(l * e, w2_ref[...], preferred_element_type=jnp.float32)
    acc += bias_ref[...]
    out_ref[...] = jnp.where(acc >= 0.0, acc, 0.2 * acc)


def _dense_layer(E, L_E, W1i, W2i, bias):
    grid = (N // ROW_BLK,)
    return pl.pallas_call(
        _dense_layer_body,
        grid=grid,
        in_specs=[
            pl.BlockSpec((ROW_BLK, EMB), lambda i: (i, 0)),
            pl.BlockSpec((ROW_BLK, EMB), lambda i: (i, 0)),
            pl.BlockSpec((EMB, EMB), lambda i: (0, 0)),
            pl.BlockSpec((EMB, EMB), lambda i: (0, 0)),
            pl.BlockSpec((1, EMB), lambda i: (0, 0)),
        ],
        out_specs=pl.BlockSpec((ROW_BLK, EMB), lambda i: (i, 0)),
        out_shape=jax.ShapeDtypeStruct((N, EMB), jnp.float32),
    )(E, L_E, W1i, W2i, bias.reshape(1, EMB))


def _norm_body(g_ref, out_ref):
    g = g_ref[...]
    nrm = jnp.sqrt(jnp.sum(g * g, axis=1, keepdims=True))
    out_ref[...] = g / jnp.maximum(nrm, 1e-12)


def _normalize_rows(G):
    m = G.shape[0]
    blk = 2048
    pad = (-m) % blk
    Gp = jnp.pad(G, ((0, pad), (0, 0)))
    out = pl.pallas_call(
        _norm_body,
        grid=((m + pad) // blk,),
        in_specs=[pl.BlockSpec((blk, EMB), lambda i: (i, 0))],
        out_specs=pl.BlockSpec((blk, EMB), lambda i: (i, 0)),
        out_shape=jax.ShapeDtypeStruct((m + pad, EMB), jnp.float32),
    )(Gp)
    return out[:m]


def kernel(user_table, item_table, age_table, sex_table, month_table, day_table,
           dow_table, W1, b1, W2, b2, lap_vals, u_id, age, sex, month, day, dow,
           pos_item, neg_item, lap_rows, lap_cols, year, node_flag):
    feats = jnp.concatenate([age_table[age], sex_table[sex], month_table[month],
                             day_table[day], dow_table[dow]], axis=1)
    upd = user_table[u_id] * (1.0 - EMB_RATIO) + feats * EMB_RATIO
    user_table = user_table.at[u_id].set(upd)
    E = jnp.concatenate([user_table, item_table], axis=0)

    raw = [E]
    for i in range(N_LAYER):
        gathered = E[lap_cols] * lap_vals[:, None]
        L_E = jax.ops.segment_sum(gathered, lap_rows, num_segments=N)
        bias = 2.0 * b1[i] + b2[i]
        E = _dense_layer(E, L_E, W1[i], W2[i], bias)
        raw.append(E)

    cat_idx = jnp.concatenate([u_id, N_USER + pos_item, N_USER + neg_item])
    g0 = raw[0][cat_idx]
    gs = [g0] + [_normalize_rows(raw[i + 1][cat_idx]) for i in range(N_LAYER)]
    allg = jnp.concatenate(gs, axis=1)
    return (allg[:B], allg[B:2 * B], allg[2 * B:])


# speedup vs baseline: 1.0259x; 1.0259x over previous
import functools

import jax
import jax.numpy as jnp
from jax.experimental import pallas as pl
from jax.experimental.pallas import tpu as pltpu

N_USER = 25000
N_ITEM = 25000
N = N_USER + N_ITEM
NNZ = 800000
EMB = 80
B = 4096
N_LAYER = 3
EMB_RATIO = 0.5

ROW_BLK = 2000


def _dense_layer_body(e_ref, l_ref, w1_ref, w2_ref, bias_ref, out_ref):
    e = e_ref[...]
    l = l_ref[...]
    acc = jnp.dot(l + e, w1_ref[...], preferred_element_type=jnp.float32)
    acc += jnp.dot(l * e, w2_ref[...], preferred_element_type=jnp.float32)
    acc += bias_ref[...]
    out_ref[...] = jnp.where(acc >= 0.0, acc, 0.2 * acc)


def _dense_layer(E, L_E, W1i, W2i, bias):
    grid = (N // ROW_BLK,)
    return pl.pallas_call(
        _dense_layer_body,
        grid=grid,
        in_specs=[
            pl.BlockSpec((ROW_BLK, EMB), lambda i: (i, 0)),
            pl.BlockSpec((ROW_BLK, EMB), lambda i: (i, 0)),
            pl.BlockSpec((EMB, EMB), lambda i: (0, 0)),
            pl.BlockSpec((EMB, EMB), lambda i: (0, 0)),
            pl.BlockSpec((1, EMB), lambda i: (0, 0)),
        ],
        out_specs=pl.BlockSpec((ROW_BLK, EMB), lambda i: (i, 0)),
        out_shape=jax.ShapeDtypeStruct((N, EMB), jnp.float32),
    )(E, L_E, W1i, W2i, bias.reshape(1, EMB))


def _norm_body(g_ref, out_ref):
    g = g_ref[...]
    nrm = jnp.sqrt(jnp.sum(g * g, axis=1, keepdims=True))
    out_ref[...] = g / jnp.maximum(nrm, 1e-12)


def _normalize_rows(G):
    m = G.shape[0]
    blk = 2048
    pad = (-m) % blk
    Gp = jnp.pad(G, ((0, pad), (0, 0)))
    out = pl.pallas_call(
        _norm_body,
        grid=((m + pad) // blk,),
        in_specs=[pl.BlockSpec((blk, EMB), lambda i: (i, 0))],
        out_specs=pl.BlockSpec((blk, EMB), lambda i: (i, 0)),
        out_shape=jax.ShapeDtypeStruct((m + pad, EMB), jnp.float32),
    )(Gp)
    return out[:m]


def kernel(user_table, item_table, age_table, sex_table, month_table, day_table,
           dow_table, W1, b1, W2, b2, lap_vals, u_id, age, sex, month, day, dow,
           pos_item, neg_item, lap_rows, lap_cols, year, node_flag):
    feats = jnp.concatenate([age_table[age], sex_table[sex], month_table[month],
                             day_table[day], dow_table[dow]], axis=1)
    upd = user_table[u_id] * (1.0 - EMB_RATIO) + feats * EMB_RATIO
    user_table = user_table.at[u_id].set(upd)
    E = jnp.concatenate([user_table, item_table], axis=0)

    raw = [E]
    for i in range(N_LAYER):
        gathered = E[lap_cols] * lap_vals[:, None]
        L_E = jax.ops.segment_sum(gathered, lap_rows, num_segments=N)
        bias = 2.0 * b1[i] + b2[i]
        E = _dense_layer(E, L_E, W1[i], W2[i], bias)
        raw.append(E)

    cat_idx = jnp.concatenate([u_id, N_USER + pos_item, N_USER + neg_item])
    g0 = raw[0][cat_idx]
    gs = [g0] + [_normalize_rows(raw[i + 1][cat_idx]) for i in range(N_LAYER)]
    allg = jnp.concatenate(gs, axis=1)
    return (allg[:B], allg[B:2 * B], allg[2 * B:])


# speedup vs baseline: 5.7811x; 5.6349x over previous
import functools

import jax
import jax.numpy as jnp
from jax import lax
from jax.experimental import pallas as pl
from jax.experimental.pallas import tpu as pltpu
from jax.experimental.pallas import tpu_sc as plsc

N_USER = 25000
N_ITEM = 25000
N = N_USER + N_ITEM
N_HALF = N // 2
NNZ = 800000
EMB = 80
B = 4096
N_LAYER = 3
EMB_RATIO = 0.5

ROW_BLK = 2000
EBLK = 512
SUBBLK = 4
NNZ_PAD = ((NNZ + EBLK - 1) // EBLK) * EBLK
N_BLOCKS = NNZ_PAD // EBLK
N_QTR = N // 4
WB_ROWS = 50
WB_CHUNKS = N_QTR // WB_ROWS
NSUB = 16


def _scalar(vec16, i):
    return lax.squeeze(lax.slice(vec16, (i,), (i + 1,)), (0,))


def _spmm_body(e_hbm, cols_hbm, rows_hbm, vals_hbm, sm_hbm, out_hbm,
               acc_sh, colsb, rowsb, valsb, gathb, zbuf, smv, sem):
    c = lax.axis_index("c")
    s = lax.axis_index("s")

    pltpu.sync_copy(sm_hbm, smv)
    sm = smv[...]

    def _zrow(i, _):
        for f in range(EMB // 16):
            zbuf[i, pl.ds(f * 16, 16)] = jnp.zeros((16,), jnp.float32)
        return 0
    lax.fori_loop(0, WB_ROWS, _zrow, 0)
    nz = (WB_CHUNKS - s + NSUB - 1) // NSUB

    for phase in range(2):
        lo = jnp.where(c == 0, _scalar(sm, 2 * phase),
                       _scalar(sm, 2 * (phase + 2)))
        hi = jnp.where(c == 0, _scalar(sm, 2 * phase + 1),
                       _scalar(sm, 2 * (phase + 2) + 1))
        rlo = (2 * c + phase) * N_QTR

        def _zchunk(i, _):
            cidx = s + i * NSUB
            pltpu.sync_copy(zbuf, acc_sh.at[pl.ds(cidx * WB_ROWS, WB_ROWS)])
            return 0
        lax.fori_loop(0, nz, _zchunk, 0)
        plsc.subcore_barrier()

        def _block(i, _):
            blk = lo + s + i * NSUB
            pltpu.sync_copy(cols_hbm.at[pl.ds(blk * SUBBLK, SUBBLK)], colsb)
            pltpu.sync_copy(rows_hbm.at[pl.ds(blk * SUBBLK, SUBBLK)], rowsb)
            pltpu.sync_copy(vals_hbm.at[pl.ds(blk * SUBBLK, SUBBLK)], valsb)

            for t in range(SUBBLK):
                for k in range(128 // 16):
                    r = rowsb[t, pl.ds(k * 16, 16)]
                    v = valsb[t, pl.ds(k * 16, 16)]
                    own = (r >= rlo) & (r < rlo + N_QTR)
                    valsb[t, pl.ds(k * 16, 16)] = jnp.where(own, v, 0.0)
                    rowsb[t, pl.ds(k * 16, 16)] = jnp.clip(r - rlo, 0,
                                                           N_QTR - 1)

            cps = [pltpu.async_copy(e_hbm.at[colsb.at[t]], gathb.at[t], sem)
                   for t in range(SUBBLK)]
            for cp in cps:
                cp.wait()

            for t in range(SUBBLK):
                def _edge(e, _):
                    v16 = valsb[t, pl.ds((e // 16) * 16, 16)]
                    bval = lax.gather(
                        v16, jnp.full((16, 1), e % 16, jnp.int32),
                        dimension_numbers=lax.GatherDimensionNumbers(
                            offset_dims=(), collapsed_slice_dims=(0,),
                            start_index_map=(0,)),
                        slice_sizes=(1,),
                        mode=lax.GatherScatterMode.PROMISE_IN_BOUNDS)
                    for f in range(EMB // 16):
                        g = gathb[t, e, pl.ds(f * 16, 16)]
                        gathb[t, e, pl.ds(f * 16, 16)] = g * bval
                    return 0
                lax.fori_loop(0, 128, _edge, 0)

            for t in range(SUBBLK):
                pltpu.sync_copy(gathb.at[t], acc_sh.at[rowsb.at[t]],
                                add=True)
            return 0

        nblk = jnp.maximum(0, (hi - lo - s + NSUB - 1) // NSUB)
        lax.fori_loop(0, nblk, _block, 0)
        plsc.subcore_barrier()

        def _wb(i, _):
            cidx = s + i * NSUB
            pltpu.sync_copy(acc_sh.at[pl.ds(cidx * WB_ROWS, WB_ROWS)],
                            out_hbm.at[pl.ds(rlo + cidx * WB_ROWS, WB_ROWS)])
            return 0
        lax.fori_loop(0, nz, _wb, 0)


_spmm_call = functools.partial(
    pl.kernel,
    out_type=jax.ShapeDtypeStruct((N, EMB), jnp.float32),
    mesh=plsc.VectorSubcoreMesh(core_axis_name="c", subcore_axis_name="s"),
    compiler_params=pltpu.CompilerParams(use_tc_tiling_on_sc=False),
    scratch_types=[
        pltpu.VMEM_SHARED((N_QTR, EMB), jnp.float32),
        pltpu.VMEM((SUBBLK, 128), jnp.int32),
        pltpu.VMEM((SUBBLK, 128), jnp.int32),
        pltpu.VMEM((SUBBLK, 128), jnp.float32),
        pltpu.VMEM((SUBBLK, 128, EMB), jnp.float32),
        pltpu.VMEM((WB_ROWS, EMB), jnp.float32),
        pltpu.VMEM((16,), jnp.int32),
        pltpu.SemaphoreType.DMA,
    ],
)(_spmm_body)


def _sc_spmm(E, cols2d, rows2d, vals2d, sm):
    return _spmm_call(E, cols2d, rows2d, vals2d, sm)


def _dense_layer_body(e_ref, l_ref, w1_ref, w2_ref, bias_ref, out_ref):
    e = e_ref[...]
    l = l_ref[...]
    acc = jnp.dot(l + e, w1_ref[...], preferred_element_type=jnp.float32)
    acc += jnp.dot(l * e, w2_ref[...], preferred_element_type=jnp.float32)
    acc += bias_ref[...]
    out_ref[...] = jnp.where(acc >= 0.0, acc, 0.2 * acc)


def _dense_layer(E, L_E, W1i, W2i, bias):
    grid = (N // ROW_BLK,)
    return pl.pallas_call(
        _dense_layer_body,
        grid=grid,
        in_specs=[
            pl.BlockSpec((ROW_BLK, EMB), lambda i: (i, 0)),
            pl.BlockSpec((ROW_BLK, EMB), lambda i: (i, 0)),
            pl.BlockSpec((EMB, EMB), lambda i: (0, 0)),
            pl.BlockSpec((EMB, EMB), lambda i: (0, 0)),
            pl.BlockSpec((1, EMB), lambda i: (0, 0)),
        ],
        out_specs=pl.BlockSpec((ROW_BLK, EMB), lambda i: (i, 0)),
        out_shape=jax.ShapeDtypeStruct((N, EMB), jnp.float32),
    )(E, L_E, W1i, W2i, bias.reshape(1, EMB))


def _norm_body(g_ref, out_ref):
    g = g_ref[...]
    nrm = jnp.sqrt(jnp.sum(g * g, axis=1, keepdims=True))
    out_ref[...] = g / jnp.maximum(nrm, 1e-12)


def _normalize_rows(G):
    m = G.shape[0]
    blk = 2048
    pad = (-m) % blk
    Gp = jnp.pad(G, ((0, pad), (0, 0)))
    out = pl.pallas_call(
        _norm_body,
        grid=((m + pad) // blk,),
        in_specs=[pl.BlockSpec((blk, EMB), lambda i: (i, 0))],
        out_specs=pl.BlockSpec((blk, EMB), lambda i: (i, 0)),
        out_shape=jax.ShapeDtypeStruct((m + pad, EMB), jnp.float32),
    )(Gp)
    return out[:m]


def kernel(user_table, item_table, age_table, sex_table, month_table, day_table,
           dow_table, W1, b1, W2, b2, lap_vals, u_id, age, sex, month, day, dow,
           pos_item, neg_item, lap_rows, lap_cols, year, node_flag):
    feats = jnp.concatenate([age_table[age], sex_table[sex], month_table[month],
                             day_table[day], dow_table[dow]], axis=1)
    upd = user_table[u_id] * (1.0 - EMB_RATIO) + feats * EMB_RATIO
    user_table = user_table.at[u_id].set(upd)
    E = jnp.concatenate([user_table, item_table], axis=0)

    pad = NNZ_PAD - NNZ
    cols2d = jnp.pad(lap_cols, (0, pad)).reshape(NNZ_PAD // 128, 128)
    rows_p = jnp.pad(lap_rows, (0, pad), constant_values=N - 1)
    rows2d = rows_p.reshape(NNZ_PAD // 128, 128)
    vals2d = jnp.pad(lap_vals, (0, pad)).reshape(NNZ_PAD // 128, 128)
    Sq = jnp.searchsorted(lap_rows, jnp.array([N_QTR, 2 * N_QTR, 3 * N_QTR],
                                              jnp.int32)).astype(jnp.int32)
    lo_q = jnp.concatenate([jnp.zeros((1,), jnp.int32), Sq // EBLK])
    hi_q = jnp.concatenate([(Sq + EBLK - 1) // EBLK,
                            jnp.full((1,), N_BLOCKS, jnp.int32)])
    sm = jnp.concatenate([jnp.stack([lo_q, hi_q], axis=1).reshape(8),
                          jnp.zeros((8,), jnp.int32)])

    raw = [E]
    for i in range(N_LAYER):
        L_E = _sc_spmm(E, cols2d, rows2d, vals2d, sm)
        bias = 2.0 * b1[i] + b2[i]
        E = _dense_layer(E, L_E, W1[i], W2[i], bias)
        raw.append(E)

    cat_idx = jnp.concatenate([u_id, N_USER + pos_item, N_USER + neg_item])
    g0 = raw[0][cat_idx]
    gs = [g0] + [_normalize_rows(raw[i + 1][cat_idx]) for i in range(N_LAYER)]
    allg = jnp.concatenate(gs, axis=1)
    return (allg[:B], allg[B:2 * B], allg[2 * B:])
